# baseline (speedup 1.0000x reference)
import jax
import jax.numpy as jnp
from jax.experimental import pallas as pl
from jax.experimental.pallas import tpu as pltpu

EMBED = 512
HIDDEN = 512
BATCH = 64
SEQ = 32
G4 = 4 * HIDDEN
GC = 2 * G4
UNROLL = 8
NW = SEQ // UNROLL


def _dotT(a, w):
    return jax.lax.dot_general(
        a, w, (((1,), (1,)), ((), ())), preferred_element_type=jnp.float32)


def _fused_kernel(tok_ref, xa_ref, xb_ref, wih0_ref, wih1_ref, whh0_ref, whh1_ref,
                  bi0_ref, bh0_ref, bi1_ref, bh1_ref,
                  out_ref, hF_ref, cF_ref,
                  wx_scr, wh_scr, bx_scr, par_scr, xg_scr, h_scr, c_scr):
    i = pl.program_id(0)

    @pl.when(i == 0)
    def _prep():
        h_scr[...] = jnp.zeros_like(h_scr)
        c_scr[...] = jnp.zeros_like(c_scr)
        wx_scr[:G4] = wih0_ref[...].astype(jnp.bfloat16)
        wx_scr[G4:] = wih1_ref[...].astype(jnp.bfloat16)
        wh_scr[:G4] = whh0_ref[...].astype(jnp.bfloat16)
        wh_scr[G4:] = whh1_ref[...].astype(jnp.bfloat16)
        bx_scr[:, :G4] = bi0_ref[...] + bh0_ref[...]
        bx_scr[:, G4:] = bi1_ref[...] + bh1_ref[...]
        par_scr[...] = (tok_ref[...] % 2).astype(jnp.float32)
        x0 = xa_ref[...].reshape(UNROLL * BATCH, EMBED).astype(jnp.bfloat16)
        xg_scr[0] = (_dotT(x0, wx_scr[...]) + bx_scr[...]).reshape(UNROLL, BATCH, GC)

    xn = xb_ref[...].reshape(UNROLL * BATCH, EMBED).astype(jnp.bfloat16)
    nxt = (_dotT(xn, wx_scr[...]) + bx_scr[...]).reshape(UNROLL, BATCH, GC)

    lane = jax.lax.broadcasted_iota(jnp.int32, (BATCH, SEQ), 1)
    h = h_scr[...]
    c = c_scr[...]
    for k in range(UNROLL):
        g = xg_scr[i % 2, k] + _dotT(h.astype(jnp.bfloat16), wh_scr[...])

        t = i * UNROLL + k
        mcol = jnp.sum(jnp.where(lane == t, par_scr[...], 0.0),
                       axis=1, keepdims=True)
        m = mcol > 0.5
        gi = jnp.where(m, g[:, 4 * HIDDEN:5 * HIDDEN], g[:, 0 * HIDDEN:1 * HIDDEN])
        gf = jnp.where(m, g[:, 5 * HIDDEN:6 * HIDDEN], g[:, 1 * HIDDEN:2 * HIDDEN])
        gg = jnp.where(m, g[:, 6 * HIDDEN:7 * HIDDEN], g[:, 2 * HIDDEN:3 * HIDDEN])
        go = jnp.where(m, g[:, 7 * HIDDEN:8 * HIDDEN], g[:, 3 * HIDDEN:4 * HIDDEN])

        c = jax.nn.sigmoid(gf) * c + jax.nn.sigmoid(gi) * jnp.tanh(gg)
        h = jax.nn.sigmoid(go) * jnp.tanh(c)
        out_ref[k] = h

    xg_scr[(i + 1) % 2] = nxt
    h_scr[...] = h
    c_scr[...] = c
    hF_ref[...] = h
    cF_ref[...] = c


def kernel(input, input_embed, W_ih0, W_hh0, b_ih0, b_hh0, W_ih1, W_hh1, b_ih1, b_hh1):
    resident = lambda shape: pl.BlockSpec(shape, lambda t: tuple(0 for _ in shape))

    out, hF, cF = pl.pallas_call(
        _fused_kernel,
        grid=(NW,),
        in_specs=[
            resident((BATCH, SEQ)),
            pl.BlockSpec((UNROLL, BATCH, EMBED), lambda i: (i, 0, 0)),
            pl.BlockSpec((UNROLL, BATCH, EMBED),
                         lambda i: (jnp.minimum(i + 1, NW - 1), 0, 0)),
            resident((G4, EMBED)),
            resident((G4, EMBED)),
            resident((G4, HIDDEN)),
            resident((G4, HIDDEN)),
            resident((1, G4)),
            resident((1, G4)),
            resident((1, G4)),
            resident((1, G4)),
        ],
        out_specs=[
            pl.BlockSpec((UNROLL, BATCH, HIDDEN), lambda i: (i, 0, 0)),
            resident((BATCH, HIDDEN)),
            resident((BATCH, HIDDEN)),
        ],
        out_shape=[
            jax.ShapeDtypeStruct((SEQ, BATCH, HIDDEN), jnp.float32),
            jax.ShapeDtypeStruct((BATCH, HIDDEN), jnp.float32),
            jax.ShapeDtypeStruct((BATCH, HIDDEN), jnp.float32),
        ],
        scratch_shapes=[
            pltpu.VMEM((GC, EMBED), jnp.bfloat16),
            pltpu.VMEM((GC, HIDDEN), jnp.bfloat16),
            pltpu.VMEM((1, GC), jnp.float32),
            pltpu.VMEM((BATCH, SEQ), jnp.float32),
            pltpu.VMEM((2, UNROLL, BATCH, GC), jnp.float32),
            pltpu.VMEM((BATCH, HIDDEN), jnp.float32),
            pltpu.VMEM((BATCH, HIDDEN), jnp.float32),
        ],
    )(input, input_embed, input_embed, W_ih0, W_ih1, W_hh0, W_hh1,
      b_ih0.reshape(1, G4), b_hh0.reshape(1, G4),
      b_ih1.reshape(1, G4), b_hh1.reshape(1, G4))

    return out, (hF, cF)


# speedup vs baseline: 1.1139x; 1.1139x over previous
import jax
import jax.numpy as jnp
from jax.experimental import pallas as pl
from jax.experimental.pallas import tpu as pltpu

EMBED = 512
HIDDEN = 512
BATCH = 64
SEQ = 32
G4 = 4 * HIDDEN
GC = 2 * G4
UNROLL = 8
NW = SEQ // UNROLL


def _dotT(a, w):
    return jax.lax.dot_general(
        a, w, (((1,), (1,)), ((), ())), preferred_element_type=jnp.float32)


def _fused_kernel(tok_ref, xa_ref, xb_ref, wih0_ref, wih1_ref, whh0_ref, whh1_ref,
                  bi0_ref, bh0_ref, bi1_ref, bh1_ref,
                  out_ref, hF_ref, cF_ref,
                  wx_scr, wh_scr, bx_scr, par_scr, xg_scr, h_scr, c_scr):
    i = pl.program_id(0)

    @pl.when(i == 0)
    def _prep():
        h_scr[...] = jnp.zeros_like(h_scr)
        c_scr[...] = jnp.zeros_like(c_scr)
        wx_scr[:G4] = wih0_ref[...].astype(jnp.bfloat16)
        wx_scr[G4:] = wih1_ref[...].astype(jnp.bfloat16)
        wh_scr[:G4] = whh0_ref[...].astype(jnp.bfloat16)
        wh_scr[G4:] = whh1_ref[...].astype(jnp.bfloat16)
        bx_scr[:, :G4] = bi0_ref[...] + bh0_ref[...]
        bx_scr[:, G4:] = bi1_ref[...] + bh1_ref[...]
        par_scr[...] = (tok_ref[...] % 2).astype(jnp.float32)
        x0 = xa_ref[...].reshape(UNROLL * BATCH, EMBED).astype(jnp.bfloat16)
        xg_scr[0] = (_dotT(x0, wx_scr[...]) + bx_scr[...]).reshape(UNROLL, BATCH, GC)

    xn = xb_ref[...].reshape(UNROLL * BATCH, EMBED).astype(jnp.bfloat16)
    xg_scr[(i + 1) % 2] = (_dotT(xn, wx_scr[...]) + bx_scr[...]).reshape(UNROLL, BATCH, GC)

    lane = jax.lax.broadcasted_iota(jnp.int32, (BATCH, SEQ), 1)
    h = h_scr[...]
    c = c_scr[...]
    for k in range(UNROLL):
        g = xg_scr[i % 2, k] + _dotT(h.astype(jnp.bfloat16), wh_scr[...])

        t = i * UNROLL + k
        mcol = jnp.sum(jnp.where(lane == t, par_scr[...], 0.0),
                       axis=1, keepdims=True)
        m = mcol > 0.5
        gi = jnp.where(m, g[:, 4 * HIDDEN:5 * HIDDEN], g[:, 0 * HIDDEN:1 * HIDDEN])
        gf = jnp.where(m, g[:, 5 * HIDDEN:6 * HIDDEN], g[:, 1 * HIDDEN:2 * HIDDEN])
        gg = jnp.where(m, g[:, 6 * HIDDEN:7 * HIDDEN], g[:, 2 * HIDDEN:3 * HIDDEN])
        go = jnp.where(m, g[:, 7 * HIDDEN:8 * HIDDEN], g[:, 3 * HIDDEN:4 * HIDDEN])

        c = jax.nn.sigmoid(gf) * c + jax.nn.sigmoid(gi) * jnp.tanh(gg)
        h = jax.nn.sigmoid(go) * jnp.tanh(c)
        out_ref[k] = h

    h_scr[...] = h
    c_scr[...] = c
    hF_ref[...] = h
    cF_ref[...] = c


def kernel(input, input_embed, W_ih0, W_hh0, b_ih0, b_hh0, W_ih1, W_hh1, b_ih1, b_hh1):
    resident = lambda shape: pl.BlockSpec(shape, lambda t: tuple(0 for _ in shape))

    out, hF, cF = pl.pallas_call(
        _fused_kernel,
        grid=(NW,),
        in_specs=[
            resident((BATCH, SEQ)),
            pl.BlockSpec((UNROLL, BATCH, EMBED), lambda i: (i, 0, 0)),
            pl.BlockSpec((UNROLL, BATCH, EMBED),
                         lambda i: (jnp.minimum(i + 1, NW - 1), 0, 0)),
            resident((G4, EMBED)),
            resident((G4, EMBED)),
            resident((G4, HIDDEN)),
            resident((G4, HIDDEN)),
            resident((1, G4)),
            resident((1, G4)),
            resident((1, G4)),
            resident((1, G4)),
        ],
        out_specs=[
            pl.BlockSpec((UNROLL, BATCH, HIDDEN), lambda i: (i, 0, 0)),
            resident((BATCH, HIDDEN)),
            resident((BATCH, HIDDEN)),
        ],
        out_shape=[
            jax.ShapeDtypeStruct((SEQ, BATCH, HIDDEN), jnp.float32),
            jax.ShapeDtypeStruct((BATCH, HIDDEN), jnp.float32),
            jax.ShapeDtypeStruct((BATCH, HIDDEN), jnp.float32),
        ],
        scratch_shapes=[
            pltpu.VMEM((GC, EMBED), jnp.bfloat16),
            pltpu.VMEM((GC, HIDDEN), jnp.bfloat16),
            pltpu.VMEM((1, GC), jnp.float32),
            pltpu.VMEM((BATCH, SEQ), jnp.float32),
            pltpu.VMEM((2, UNROLL, BATCH, GC), jnp.float32),
            pltpu.VMEM((BATCH, HIDDEN), jnp.float32),
            pltpu.VMEM((BATCH, HIDDEN), jnp.float32),
        ],
    )(input, input_embed, input_embed, W_ih0, W_ih1, W_hh0, W_hh1,
      b_ih0.reshape(1, G4), b_hh0.reshape(1, G4),
      b_ih1.reshape(1, G4), b_hh1.reshape(1, G4))

    return out, (hF, cF)


# speedup vs baseline: 1.1604x; 1.0417x over previous
import jax
import jax.numpy as jnp
from jax.experimental import pallas as pl
from jax.experimental.pallas import tpu as pltpu

EMBED = 512
HIDDEN = 512
BATCH = 64
SEQ = 32
G4 = 4 * HIDDEN
GC = 2 * G4
UNROLL = 16


def _dotT(a, w):
    return jax.lax.dot_general(
        a, w, (((1,), (1,)), ((), ())), preferred_element_type=jnp.float32)


def _fused_kernel(tok_ref, x_ref, wih0_ref, wih1_ref, whh0_ref, whh1_ref,
                  bi0_ref, bh0_ref, bi1_ref, bh1_ref,
                  out_ref, hF_ref, cF_ref,
                  wx_scr, wh_scr, bx_scr, par_scr, xg_scr, h_scr, c_scr):
    i = pl.program_id(0)

    @pl.when(i == 0)
    def _prep():
        h_scr[...] = jnp.zeros_like(h_scr)
        c_scr[...] = jnp.zeros_like(c_scr)
        wx_scr[:G4] = wih0_ref[...].astype(jnp.bfloat16)
        wx_scr[G4:] = wih1_ref[...].astype(jnp.bfloat16)
        wh_scr[:G4] = whh0_ref[...].astype(jnp.bfloat16)
        wh_scr[G4:] = whh1_ref[...].astype(jnp.bfloat16)
        bx_scr[:, :G4] = bi0_ref[...] + bh0_ref[...]
        bx_scr[:, G4:] = bi1_ref[...] + bh1_ref[...]
        par_scr[...] = (tok_ref[...] % 2).astype(jnp.float32)

    x = x_ref[...].reshape(UNROLL * BATCH, EMBED).astype(jnp.bfloat16)
    xg_scr[...] = (_dotT(x, wx_scr[...]) + bx_scr[...]).reshape(UNROLL, BATCH, GC)

    lane = jax.lax.broadcasted_iota(jnp.int32, (BATCH, SEQ), 1)
    h = h_scr[...]
    c = c_scr[...]
    for k in range(UNROLL):
        g = xg_scr[k] + _dotT(h.astype(jnp.bfloat16), wh_scr[...])

        t = i * UNROLL + k
        mcol = jnp.sum(jnp.where(lane == t, par_scr[...], 0.0),
                       axis=1, keepdims=True)
        m = mcol > 0.5
        gi = jnp.where(m, g[:, 4 * HIDDEN:5 * HIDDEN], g[:, 0 * HIDDEN:1 * HIDDEN])
        gf = jnp.where(m, g[:, 5 * HIDDEN:6 * HIDDEN], g[:, 1 * HIDDEN:2 * HIDDEN])
        gg = jnp.where(m, g[:, 6 * HIDDEN:7 * HIDDEN], g[:, 2 * HIDDEN:3 * HIDDEN])
        go = jnp.where(m, g[:, 7 * HIDDEN:8 * HIDDEN], g[:, 3 * HIDDEN:4 * HIDDEN])

        c = jax.nn.sigmoid(gf) * c + jax.nn.sigmoid(gi) * jnp.tanh(gg)
        h = jax.nn.sigmoid(go) * jnp.tanh(c)
        out_ref[k] = h

    h_scr[...] = h
    c_scr[...] = c
    hF_ref[...] = h
    cF_ref[...] = c


def kernel(input, input_embed, W_ih0, W_hh0, b_ih0, b_hh0, W_ih1, W_hh1, b_ih1, b_hh1):
    resident = lambda shape: pl.BlockSpec(shape, lambda t: tuple(0 for _ in shape))

    out, hF, cF = pl.pallas_call(
        _fused_kernel,
        grid=(SEQ // UNROLL,),
        in_specs=[
            resident((BATCH, SEQ)),
            pl.BlockSpec((UNROLL, BATCH, EMBED), lambda i: (i, 0, 0)),
            resident((G4, EMBED)),
            resident((G4, EMBED)),
            resident((G4, HIDDEN)),
            resident((G4, HIDDEN)),
            resident((1, G4)),
            resident((1, G4)),
            resident((1, G4)),
            resident((1, G4)),
        ],
        out_specs=[
            pl.BlockSpec((UNROLL, BATCH, HIDDEN), lambda i: (i, 0, 0)),
            resident((BATCH, HIDDEN)),
            resident((BATCH, HIDDEN)),
        ],
        out_shape=[
            jax.ShapeDtypeStruct((SEQ, BATCH, HIDDEN), jnp.float32),
            jax.ShapeDtypeStruct((BATCH, HIDDEN), jnp.float32),
            jax.ShapeDtypeStruct((BATCH, HIDDEN), jnp.float32),
        ],
        scratch_shapes=[
            pltpu.VMEM((GC, EMBED), jnp.bfloat16),
            pltpu.VMEM((GC, HIDDEN), jnp.bfloat16),
            pltpu.VMEM((1, GC), jnp.float32),
            pltpu.VMEM((BATCH, SEQ), jnp.float32),
            pltpu.VMEM((UNROLL, BATCH, GC), jnp.float32),
            pltpu.VMEM((BATCH, HIDDEN), jnp.float32),
            pltpu.VMEM((BATCH, HIDDEN), jnp.float32),
        ],
    )(input, input_embed, W_ih0, W_ih1, W_hh0, W_hh1,
      b_ih0.reshape(1, G4), b_hh0.reshape(1, G4),
      b_ih1.reshape(1, G4), b_hh1.reshape(1, G4))

    return out, (hF, cF)


# speedup vs baseline: 1.1618x; 1.0012x over previous
import jax
import jax.numpy as jnp
from jax.experimental import pallas as pl
from jax.experimental.pallas import tpu as pltpu

EMBED = 512
HIDDEN = 512
BATCH = 64
SEQ = 32
G4 = 4 * HIDDEN
GC = 2 * G4
UNROLL = 16


def _dotT(a, w):
    return jax.lax.dot_general(
        a, w, (((1,), (1,)), ((), ())), preferred_element_type=jnp.float32)


def _sel(m, z, lo):
    return jnp.where(m, z[:, G4 + lo:G4 + lo + HIDDEN], z[:, lo:lo + HIDDEN])


def _fused_kernel(tok_ref, x_ref, wih0_ref, wih1_ref, whh0_ref, whh1_ref,
                  bi0_ref, bh0_ref, bi1_ref, bh1_ref,
                  out_ref, hF_ref, cF_ref,
                  wx_scr, wh_scr, bx_scr, par_scr, xg_scr, h_scr, c_scr):
    i = pl.program_id(0)

    @pl.when(i == 0)
    def _prep():
        h_scr[...] = jnp.zeros_like(h_scr)
        c_scr[...] = jnp.zeros_like(c_scr)
        wx_scr[:G4] = wih0_ref[...].astype(jnp.bfloat16)
        wx_scr[G4:] = wih1_ref[...].astype(jnp.bfloat16)
        wh_scr[:G4] = whh0_ref[...].astype(jnp.bfloat16)
        wh_scr[G4:] = whh1_ref[...].astype(jnp.bfloat16)
        bx_scr[:, :G4] = bi0_ref[...] + bh0_ref[...]
        bx_scr[:, G4:] = bi1_ref[...] + bh1_ref[...]
        par_scr[...] = (tok_ref[...] % 2).astype(jnp.float32)

    lane = jax.lax.broadcasted_iota(jnp.int32, (BATCH, SEQ), 1)

    def mask(t):
        mcol = jnp.sum(jnp.where(lane == t, par_scr[...], 0.0),
                       axis=1, keepdims=True)
        return mcol > 0.5

    x = x_ref[...].reshape(UNROLL * BATCH, EMBED).astype(jnp.bfloat16)
    xg = (_dotT(x, wx_scr[...]) + bx_scr[...]).reshape(UNROLL, BATCH, GC)
    for k in range(UNROLL):
        mk = mask(i * UNROLL + k)
        xg_scr[k] = jnp.concatenate(
            [_sel(mk, xg[k], 0), _sel(mk, xg[k], HIDDEN),
             _sel(mk, xg[k], 2 * HIDDEN), _sel(mk, xg[k], 3 * HIDDEN)], axis=1)

    h = h_scr[...]
    c = c_scr[...]
    for k in range(UNROLL):
        d = _dotT(h.astype(jnp.bfloat16), wh_scr[...])
        m = mask(i * UNROLL + k)
        xk = xg_scr[k]
        gi = xk[:, 0 * HIDDEN:1 * HIDDEN] + _sel(m, d, 0)
        gf = xk[:, 1 * HIDDEN:2 * HIDDEN] + _sel(m, d, HIDDEN)
        gg = xk[:, 2 * HIDDEN:3 * HIDDEN] + _sel(m, d, 2 * HIDDEN)
        go = xk[:, 3 * HIDDEN:4 * HIDDEN] + _sel(m, d, 3 * HIDDEN)

        c = jax.nn.sigmoid(gf) * c + jax.nn.sigmoid(gi) * jnp.tanh(gg)
        h = jax.nn.sigmoid(go) * jnp.tanh(c)
        out_ref[k] = h

    h_scr[...] = h
    c_scr[...] = c
    hF_ref[...] = h
    cF_ref[...] = c


def kernel(input, input_embed, W_ih0, W_hh0, b_ih0, b_hh0, W_ih1, W_hh1, b_ih1, b_hh1):
    resident = lambda shape: pl.BlockSpec(shape, lambda t: tuple(0 for _ in shape))

    out, hF, cF = pl.pallas_call(
        _fused_kernel,
        grid=(SEQ // UNROLL,),
        in_specs=[
            resident((BATCH, SEQ)),
            pl.BlockSpec((UNROLL, BATCH, EMBED), lambda i: (i, 0, 0)),
            resident((G4, EMBED)),
            resident((G4, EMBED)),
            resident((G4, HIDDEN)),
            resident((G4, HIDDEN)),
            resident((1, G4)),
            resident((1, G4)),
            resident((1, G4)),
            resident((1, G4)),
        ],
        out_specs=[
            pl.BlockSpec((UNROLL, BATCH, HIDDEN), lambda i: (i, 0, 0)),
            resident((BATCH, HIDDEN)),
            resident((BATCH, HIDDEN)),
        ],
        out_shape=[
            jax.ShapeDtypeStruct((SEQ, BATCH, HIDDEN), jnp.float32),
            jax.ShapeDtypeStruct((BATCH, HIDDEN), jnp.float32),
            jax.ShapeDtypeStruct((BATCH, HIDDEN), jnp.float32),
        ],
        scratch_shapes=[
            pltpu.VMEM((GC, EMBED), jnp.bfloat16),
            pltpu.VMEM((GC, HIDDEN), jnp.bfloat16),
            pltpu.VMEM((1, GC), jnp.float32),
            pltpu.VMEM((BATCH, SEQ), jnp.float32),
            pltpu.VMEM((UNROLL, BATCH, G4), jnp.float32),
            pltpu.VMEM((BATCH, HIDDEN), jnp.float32),
            pltpu.VMEM((BATCH, HIDDEN), jnp.float32),
        ],
    )(input, input_embed, W_ih0, W_ih1, W_hh0, W_hh1,
      b_ih0.reshape(1, G4), b_hh0.reshape(1, G4),
      b_ih1.reshape(1, G4), b_hh1.reshape(1, G4))

    return out, (hF, cF)
